# baseline (speedup 1.0000x reference)
import functools

import jax
import jax.numpy as jnp
from jax import lax
from jax.experimental import pallas as pl
from jax.experimental.pallas import tpu as pltpu
from jax.experimental.pallas import tpu_sc as plsc

HEADS_ = 8
POINTS_ = 4


def _val_proj_body(YB, GY, Wd, da_ref, db_ref, w_ref, b_ref, o_ref):
    dh = o_ref.shape[1] // 2
    bias = b_ref[...][None, :]
    wv = w_ref[...].astype(jnp.bfloat16)

    def proj(col):
        return lax.dot_general(col.astype(jnp.bfloat16), wv,
                               (((0,), (0,)), ((), ())),
                               preferred_element_type=jnp.float32) + bias

    va = [proj(da_ref[0, :, yy, :]) for yy in range(YB)]
    is_last = pl.program_id(1) == GY - 1
    edge_col = jnp.where(is_last, da_ref[0, :, YB - 1, :], db_ref[0, :, 0, :])
    va.append(proj(edge_col))

    def pack_x_pair(v):
        vb = lax.bitcast_convert_type(v.astype(jnp.bfloat16), jnp.uint16)
        lo = vb.astype(jnp.int32)
        sh = jnp.concatenate([lo[1:], lo[-1:]], axis=0)
        return lo | lax.shift_left(sh, 16)

    pk = [pack_x_pair(v) for v in va]
    for yy in range(YB):
        for h in range(HEADS_):
            rb = (yy * HEADS_ + h) * Wd
            o_ref[pl.ds(rb, Wd), pl.ds(0, dh)] = pk[yy][:, h * dh:(h + 1) * dh]
            o_ref[pl.ds(rb, Wd), pl.ds(dh, dh)] = \
                pk[yy + 1][:, h * dh:(h + 1) * dh]


def _val_proj(dense4, w_val, b_val):
    B, C, Hd, Wd = dense4.shape
    YB = 8
    GY = Hd // YB
    return pl.pallas_call(
        functools.partial(_val_proj_body, YB, GY, Wd),
        grid=(B, GY),
        in_specs=[
            pl.BlockSpec((1, C, YB, Wd), lambda b, i: (b, 0, i, 0)),
            pl.BlockSpec((1, C, YB, Wd),
                         lambda b, i: (b, 0, jnp.minimum(i + 1, GY - 1), 0)),
            pl.BlockSpec((C, C), lambda b, i: (0, 0)),
            pl.BlockSpec((C,), lambda b, i: (0,)),
        ],
        out_specs=pl.BlockSpec((YB * HEADS_ * Wd, 2 * (C // HEADS_)),
                               lambda b, i: (b * GY + i, 0)),
        out_shape=jax.ShapeDtypeStruct(
            (B * Hd * HEADS_ * Wd, 2 * (C // HEADS_)), jnp.int32),
    )(dense4, dense4, w_val, b_val)


def _addr_body(Hd, Wd, s_ref, x_ref, y_ref, b_ref, wo_ref, bo_ref,
               wa_ref, ba_ref, idx_ref, w_ref):
    s = s_ref[...]
    offm = lax.dot_general(s, wo_ref[...], (((1,), (0,)), ((), ())),
                           preferred_element_type=jnp.float32) + bo_ref[...][None, :]
    attn = lax.dot_general(s, wa_ref[...], (((1,), (0,)), ((), ())),
                           preferred_element_type=jnp.float32) + ba_ref[...][None, :]
    a = [attn[:, p * 8:(p + 1) * 8] for p in range(POINTS_)]
    m = jnp.maximum(jnp.maximum(a[0], a[1]), jnp.maximum(a[2], a[3]))
    e = [jnp.exp(v - m) for v in a]
    ssum = e[0] + e[1] + e[2] + e[3]
    aw = [v / ssum for v in e]

    xq = x_ref[...].astype(jnp.float32)
    yq = y_ref[...].astype(jnp.float32)
    bq = b_ref[...]
    TN = s.shape[0]
    h_arr = lax.broadcasted_iota(jnp.int32, (TN, 8), 1)
    ref_x = xq / Hd
    ref_y = yq / Wd
    idx_parts, wa_p, wb_p, wc_p, wd_p = [], [], [], [], []
    for p in range(POINTS_):
        off_x = offm[:, p * 8:(p + 1) * 8]
        off_y = offm[:, 32 + p * 8:32 + (p + 1) * 8]
        ix = (ref_x + off_x / Wd) * Wd - 0.5
        iy = (ref_y + off_y / Hd) * Hd - 0.5
        x0 = jnp.floor(ix)
        y0 = jnp.floor(iy)
        wx1 = ix - x0
        wx0 = 1.0 - wx1
        wy1 = iy - y0
        wy0 = 1.0 - wy1
        inb_y0 = ((y0 >= 0) & (y0 <= Hd - 1)).astype(jnp.float32)
        inb_y1 = ((y0 + 1.0 >= 0) & (y0 + 1.0 <= Hd - 1)).astype(jnp.float32)
        fy0 = wy0 * inb_y0 + wy1 * (y0 == -1.0).astype(jnp.float32)
        fy1 = wy1 * inb_y1 * (y0 >= 0).astype(jnp.float32)
        inb_x0 = ((x0 >= 0) & (x0 <= Wd - 1)).astype(jnp.float32)
        inb_x1 = ((x0 + 1.0 >= 0) & (x0 + 1.0 <= Wd - 1)).astype(jnp.float32)
        fx0 = wx0 * inb_x0 + wx1 * (x0 == -1.0).astype(jnp.float32)
        fx1 = wx1 * inb_x1 * (x0 >= 0).astype(jnp.float32)
        ybase = jnp.clip(y0, 0, Hd - 1).astype(jnp.int32)
        xbase = jnp.clip(x0, 0, Wd - 1).astype(jnp.int32)
        idx_parts.append(((bq * Hd + ybase) * HEADS_ + h_arr) * Wd + xbase)
        wa_p.append(aw[p] * fx0 * fy0)
        wb_p.append(aw[p] * fx1 * fy0)
        wc_p.append(aw[p] * fx0 * fy1)
        wd_p.append(aw[p] * fx1 * fy1)
    idx_ref[...] = jnp.concatenate(idx_parts, axis=1)
    w_ref[...] = jnp.concatenate(wa_p + wb_p + wc_p + wd_p, axis=1)


def _addresses(sparse, xcol, ycol, bcol, w_off2, b_off2, w_attn2, b_attn2,
               Hd, Wd):
    N, C = sparse.shape
    TN = 1000
    grid = N // TN
    return pl.pallas_call(
        functools.partial(_addr_body, Hd, Wd),
        grid=(grid,),
        in_specs=[
            pl.BlockSpec((TN, C), lambda i: (i, 0)),
            pl.BlockSpec((TN, 1), lambda i: (i, 0)),
            pl.BlockSpec((TN, 1), lambda i: (i, 0)),
            pl.BlockSpec((TN, 1), lambda i: (i, 0)),
            pl.BlockSpec((C, 64), lambda i: (0, 0)),
            pl.BlockSpec((64,), lambda i: (0,)),
            pl.BlockSpec((C, 32), lambda i: (0, 0)),
            pl.BlockSpec((32,), lambda i: (0,)),
        ],
        out_specs=[
            pl.BlockSpec((TN, 32), lambda i: (i, 0)),
            pl.BlockSpec((TN, 128), lambda i: (i, 0)),
        ],
        out_shape=[
            jax.ShapeDtypeStruct((N, 32), jnp.int32),
            jax.ShapeDtypeStruct((N, 128), jnp.float32),
        ],
    )(sparse, xcol, ycol, bcol, w_off2, b_off2, w_attn2, b_attn2)


def _bf16_pair(words):
    even = lax.bitcast_convert_type(
        lax.shift_left(words, 16), jnp.float32)
    odd = lax.bitcast_convert_type(
        jnp.bitwise_and(words, jnp.int32(-65536)), jnp.float32)
    return even, odd


def _splat_lane(vec16, lane):
    idx = jnp.full((16, 1), lane, jnp.int32)
    return lax.gather(
        vec16, idx,
        dimension_numbers=lax.GatherDimensionNumbers(
            offset_dims=(), collapsed_slice_dims=(0,), start_index_map=(0,)),
        slice_sizes=(1,),
        mode=lax.GatherScatterMode.PROMISE_IN_BOUNDS)


def _gather_combine(table, cidx, cw, Np, dh):
    NW = 32
    NQW = Np // NW
    Q = 8
    RPQ = 32
    R = Q * RPQ
    NCH = NQW // Q

    mesh = plsc.VectorSubcoreMesh(core_axis_name="c", subcore_axis_name="s")

    @functools.partial(
        pl.kernel, mesh=mesh,
        out_type=jax.ShapeDtypeStruct((Np, HEADS_ * dh), jnp.float32),
        scratch_types=[
            pltpu.VMEM((Q, RPQ), jnp.int32),
            pltpu.VMEM((Q, 4 * RPQ), jnp.float32),
            pltpu.VMEM((R, 2 * dh), jnp.int32),
            pltpu.VMEM((Q, HEADS_ * dh), jnp.float32),
            pltpu.SemaphoreType.DMA,
        ],
    )
    def sc_kernel(table_hbm, idx_hbm, w_hbm, out_hbm, idx_v, w_v, rows_v,
                  out_v, gsem):
        wid = lax.axis_index("s") * 2 + lax.axis_index("c")
        qw0 = wid * NQW

        def step(g, carry):
            q0 = qw0 + g * Q
            pltpu.sync_copy(idx_hbm.at[pl.ds(q0, Q)], idx_v)
            pltpu.sync_copy(w_hbm.at[pl.ds(q0, Q)], w_v)
            handles = [
                pltpu.async_copy(table_hbm.at[idx_v.at[qi]],
                                 rows_v.at[pl.ds(qi * RPQ, RPQ)], gsem)
                for qi in range(Q)]
            for hd in handles:
                hd.wait()

            def q_body(qi, cq):
                base = qi * RPQ

                def jj_body(jj, acc):
                    wb = jj * 16
                    w16a = w_v[qi, pl.ds(wb, 16)]
                    w16b = w_v[qi, pl.ds(RPQ + wb, 16)]
                    w16c = w_v[qi, pl.ds(2 * RPQ + wb, 16)]
                    w16d = w_v[qi, pl.ds(3 * RPQ + wb, 16)]
                    acc = list(acc)
                    for k in range(2):
                        for h in range(HEADS_):
                            lane = k * 8 + h
                            r = base + wb + lane
                            wa = _splat_lane(w16a, lane)
                            wb_ = _splat_lane(w16b, lane)
                            wc = _splat_lane(w16c, lane)
                            wd = _splat_lane(w16d, lane)
                            for gg in range(4):
                                a, b = _bf16_pair(
                                    rows_v[r, pl.ds(gg * 16, 16)])
                                c, d = _bf16_pair(
                                    rows_v[r, pl.ds(dh + gg * 16, 16)])
                                acc[h * 4 + gg] = acc[h * 4 + gg] + \
                                    wa * a + wb_ * b + wc * c + wd * d
                    return tuple(acc)

                acc0 = tuple(jnp.zeros((16,), jnp.float32)
                             for _ in range(HEADS_ * 4))
                acc = lax.fori_loop(0, RPQ // 16, jj_body, acc0)
                for h in range(HEADS_):
                    for gg in range(4):
                        out_v[qi, pl.ds(h * dh + gg * 16, 16)] = \
                            acc[h * 4 + gg]
                return cq

            lax.fori_loop(0, Q, q_body, 0)
            pltpu.sync_copy(out_v, out_hbm.at[pl.ds(q0, Q)])
            return carry

        lax.fori_loop(0, NCH, step, 0)

    return sc_kernel(table, cidx, cw)


def _out_proj_body(a_ref, w_ref, b_ref, s_ref, o_ref):
    o_ref[...] = s_ref[...] + lax.dot_general(
        a_ref[...], w_ref[...], (((1,), (0,)), ((), ())),
        preferred_element_type=jnp.float32) + b_ref[...][None, :]


def _out_proj(agg, w_out, b_out, sparse):
    N, C = sparse.shape
    TN = 1000
    return pl.pallas_call(
        _out_proj_body,
        grid=(N // TN,),
        in_specs=[
            pl.BlockSpec((TN, C), lambda i: (i, 0)),
            pl.BlockSpec((C, C), lambda i: (0, 0)),
            pl.BlockSpec((C,), lambda i: (0,)),
            pl.BlockSpec((TN, C), lambda i: (i, 0)),
        ],
        out_specs=pl.BlockSpec((TN, C), lambda i: (i, 0)),
        out_shape=jax.ShapeDtypeStruct((N, C), jnp.float32),
    )(agg, w_out, b_out, sparse)


def kernel(sparse_features, voxel_batch_idx, voxel_xy, dense_tensor,
           W_val, b_val, W_off, b_off, W_attn, b_attn, W_out, b_out):
    B, C, Hd, Wd = dense_tensor.shape
    N = sparse_features.shape[0]
    HW = Hd * Wd
    dh = C // HEADS_

    table = _val_proj(dense_tensor, W_val, b_val)

    W_off2 = W_off.reshape(C, HEADS_, POINTS_, 2).transpose(0, 3, 2, 1).reshape(C, 64)
    b_off2 = b_off.reshape(HEADS_, POINTS_, 2).transpose(2, 1, 0).reshape(64)
    W_attn2 = W_attn.reshape(C, HEADS_, POINTS_).transpose(0, 2, 1).reshape(C, 32)
    b_attn2 = b_attn.reshape(HEADS_, POINTS_).transpose(1, 0).reshape(32)
    xcol = voxel_xy[:, 0:1].astype(jnp.int32)
    ycol = voxel_xy[:, 1:2].astype(jnp.int32)
    bcol = voxel_batch_idx[:, None].astype(jnp.int32)
    cidx, cw = _addresses(sparse_features, xcol, ycol, bcol,
                          W_off2, b_off2, W_attn2, b_attn2, Hd, Wd)

    Np = ((N + 255) // 256) * 256
    cidx_p = jnp.pad(cidx, ((0, Np - N), (0, 0)))
    cw_p = jnp.pad(cw, ((0, Np - N), (0, 0)))
    agg = _gather_combine(table, cidx_p, cw_p, Np, dh)

    return _out_proj(agg, W_out, b_out, sparse_features)
